# baseline (speedup 1.0000x reference)
import numpy as np
import jax
import jax.numpy as jnp
from jax import lax
from jax.experimental import pallas as pl
from jax.experimental.pallas import tpu as pltpu
from jax.experimental.pallas import tpu_sc as plsc

_NUM_LEVELS = 16
_FEAT = 2
_B = 262144
_HS = 2 ** 19
_P2 = 2654435761
_P3 = 805459861

_P2_I32 = np.int32(np.uint32(_P2 & 0xFFFFFFFF).astype(np.int64) - (1 << 32))
_P3_I32 = np.int32(_P3)
_A2 = np.int32(_P2 >> 16)
_B2 = np.int32(_P2 & 0xFFFF)
_A3 = np.int32(_P3 >> 16)
_B3 = np.int32(_P3 & 0xFFFF)
_MININT = np.int32(-2147483648)
_LOW31 = np.int32(0x7FFFFFFF)


def _level_constants():
    b = np.exp2(np.log2(np.float32(2048) / np.float32(16)) / np.float32(15))
    res = (np.float32(16) * (np.float32(b) ** np.arange(16, dtype=np.float32))).astype(np.int32)
    out = []
    for r in res:
        r = int(r)
        m = min(r ** 3, _HS)
        out.append((r, m))
    return out


_LEVELS = _level_constants()
_OFF4 = []
_acc = 0
for _r, _m in _LEVELS:
    _OFF4.append(_acc)
    _acc += _m // 4
_TOT4 = _acc

_MASK19 = np.int32(2 ** 19 - 1)
_CST_I = np.zeros((10, 3, 16), np.int32)
_CST_F = np.zeros((10, 16), np.float32)
for _k, _l in enumerate(range(6, 16)):
    _res = _LEVELS[_l][0]
    _CST_I[_k, 0, :] = _res - 1
    _CST_I[_k, 1, :] = _res
    _CST_I[_k, 2, :] = _OFF4[_l]
    _CST_F[_k, :] = np.float32(_res)
_CST_I = _CST_I.reshape(-1)
_CST_F = _CST_F.reshape(-1)

_N = 512
_NCH = _N // 16
_NJ = _N // 8
_IDX_CHUNK = 128
_NDMA = 8 * _N // _IDX_CHUNK


def _prod64(v, a16, b16):
    u = v * a16
    w = v * b16
    lo = (u << 16) + w
    carry = jnp.where((lo ^ _MININT) < (w ^ _MININT), np.int32(1), np.int32(0))
    hi = (u >> 16) + carry
    return hi, lo


def _body(x_ref, y_ref, z_ref, *rest):
    tables = rest[:_NUM_LEVELS]
    cst_i_hbm = rest[_NUM_LEVELS]
    cst_f_hbm = rest[_NUM_LEVELS + 1]
    out_ref = rest[_NUM_LEVELS + 2]
    (packed, rp_a, rp_b, ci_ref, cf_ref, xn_ref, fr_a, fr_b, idx_a, idx_b,
     sub_a, sub_b, rows_a, rows_b, out_buf, sem_a, sem_b) = rest[_NUM_LEVELS + 3:]

    nw = 32
    wid = (lax.axis_index("s") * np.int32(2) + lax.axis_index("c")).astype(jnp.int32)
    per_w = _B // nw
    nblocks = per_w // _N
    i32 = np.int32

    iota = lax.iota(jnp.int32, 16)
    ioh = iota >> 1
    cid = iota & 1

    sid = lax.axis_index("s").astype(jnp.int32)
    for l in range(_NUM_LEVELS):
        _, m = _LEVELS[l]
        R = m // 4
        nch = -(-R // (16 * 128))

        def rp_body(c, carry, l=l, R=R, nch=nch):
            g = sid * i32(nch) + c
            row0 = jnp.minimum(g * i32(128), i32(R - 128))

            def rp_inner(i, cc):
                base = i * i32(16)
                fl = base + iota
                v = plsc.load_gather(rp_a, [fl >> 1, fl & i32(1)])
                plsc.store_scatter(rp_b, [fl >> 3, fl & i32(7)], v)
                return cc

            pltpu.sync_copy(tables[l].at[pl.ds(row0 * i32(4), 512)], rp_a)
            lax.fori_loop(i32(0), i32(64), rp_inner, i32(0))
            pltpu.sync_copy(rp_b, packed.at[pl.ds(i32(_OFF4[l]) + row0, 128)])
            return carry

        lax.fori_loop(i32(0), i32(nch), rp_body, i32(0))
    plsc.subcore_barrier()

    pltpu.sync_copy(cst_i_hbm, ci_ref)
    pltpu.sync_copy(cst_f_hbm, cf_ref)

    def pass1(l, fr_ref, idx_ref, sub_ref):
        res, m = _LEVELS[l]
        pow2 = (m & (m - 1)) == 0
        res_f = np.float32(res)
        res_m1 = np.int32(res - 1)
        res_i = np.int32(res)
        if not pow2:
            r32 = np.int32((1 << 32) % m)
            r31 = np.int32((1 << 31) % m)
            m_i = np.int32(m)
        mask = np.int32(m - 1)

        def c_body(ci, carry):
            off = ci * i32(16)
            los = []
            his = []
            for d in range(3):
                f = xn_ref[pl.ds(i32(d * _N) + off, 16)]
                xg = f * res_f
                xi = xg.astype(jnp.int32)
                fr_ref[pl.ds(i32(d * _N) + off, 16)] = xg - xi.astype(jnp.float32)
                vf = jnp.minimum(xi, res_m1)
                vc = jnp.minimum(xi + 1, res_i)
                if d == 0:
                    los.append((vf, vc))
                    his.append(None)
                else:
                    a16 = _A2 if d == 1 else _A3
                    b16 = _B2 if d == 1 else _B3
                    p32 = _P2_I32 if d == 1 else _P3_I32
                    if pow2:
                        los.append((vf * p32, vc * p32))
                        his.append(None)
                    else:
                        hf, lf = _prod64(vf, a16, b16)
                        hc, lc = _prod64(vc, a16, b16)
                        los.append((lf, lc))
                        his.append((hf, hc))
            for cn in range(8):
                sx, sy, sz = cn & 1, (cn >> 1) & 1, (cn >> 2) & 1
                lo = los[0][sx] ^ los[1][sy] ^ los[2][sz]
                if pow2:
                    h = lo & mask
                else:
                    hi = his[1][sy] ^ his[2][sz]
                    t = hi * r32 + lax.rem(lo & _LOW31, m_i)
                    t = t + jnp.where(lo < 0, r31, np.int32(0))
                    h = lax.rem(t, m_i)
                row = i32(cn * 4) + (ci >> 3)
                col = (ci & i32(7)) * i32(16)
                idx_ref[row, pl.ds(col, 16)] = (h >> 2) + i32(_OFF4[l])
                sub_ref[pl.ds(i32(cn * _N) + off, 16)] = (h & i32(3)) * i32(2)
            return carry

        lax.fori_loop(i32(0), i32(_NCH), c_body, i32(0))

    def pass1_u(kq, fr_ref, idx_ref, sub_ref):
        resf = cf_ref[pl.ds(kq * i32(16), 16)]
        cb = kq * i32(48)
        resm1 = ci_ref[pl.ds(cb, 16)]
        resi = ci_ref[pl.ds(cb + i32(16), 16)]
        off4 = ci_ref[pl.ds(cb + i32(32), 16)]

        def c_body(ci, carry):
            off = ci * i32(16)
            los = []
            for d in range(3):
                f = xn_ref[pl.ds(i32(d * _N) + off, 16)]
                xg = f * resf
                xi = xg.astype(jnp.int32)
                fr_ref[pl.ds(i32(d * _N) + off, 16)] = xg - xi.astype(jnp.float32)
                vf = jnp.minimum(xi, resm1)
                vc = jnp.minimum(xi + 1, resi)
                if d == 0:
                    los.append((vf, vc))
                else:
                    p32 = _P2_I32 if d == 1 else _P3_I32
                    los.append((vf * p32, vc * p32))
            for cn in range(8):
                sx, sy, sz = cn & 1, (cn >> 1) & 1, (cn >> 2) & 1
                lo = los[0][sx] ^ los[1][sy] ^ los[2][sz]
                h = lo & _MASK19
                row = i32(cn * 4) + (ci >> 3)
                col = (ci & i32(7)) * i32(16)
                idx_ref[row, pl.ds(col, 16)] = (h >> 2) + off4
                sub_ref[pl.ds(i32(cn * _N) + off, 16)] = (h & i32(3)) * i32(2)
            return carry

        lax.fori_loop(i32(0), i32(_NCH), c_body, i32(0))

    def fire(idx_ref, rows_ref, sem):
        def f_body(i, carry):
            pltpu.async_copy(
                packed.at[idx_ref.at[i]],
                rows_ref.at[pl.ds(i * i32(_IDX_CHUNK), _IDX_CHUNK)],
                sem,
            )
            return carry

        lax.fori_loop(i32(0), i32(_NDMA), f_body, i32(0))

    def drain(idx_ref, rows_ref, sem):
        def d_body(i, carry):
            pltpu.make_async_copy(
                packed.at[idx_ref.at[i]],
                rows_ref.at[pl.ds(i * i32(_IDX_CHUNK), _IDX_CHUNK)],
                sem,
            ).wait()
            return carry

        lax.fori_loop(i32(0), i32(_NDMA), d_body, i32(0))

    def pass2(colv, fr_ref, sub_ref, rows_ref):
        def j_body(j, carry):
            r0 = j * i32(8) + ioh
            fx = plsc.load_gather(fr_ref, [r0])
            fy = plsc.load_gather(fr_ref, [r0 + np.int32(_N)])
            fz = plsc.load_gather(fr_ref, [r0 + np.int32(2 * _N)])
            rv = []
            for cn in range(8):
                rc = r0 + np.int32(cn * _N)
                sub = plsc.load_gather(sub_ref, [rc])
                rv.append(plsc.load_gather(rows_ref, [rc, sub + cid]))
            a0 = rv[0] + fx * (rv[1] - rv[0])
            a1 = rv[2] + fx * (rv[3] - rv[2])
            a2 = rv[4] + fx * (rv[5] - rv[4])
            a3 = rv[6] + fx * (rv[7] - rv[6])
            b0 = a0 + fy * (a1 - a0)
            b1 = a2 + fy * (a3 - a2)
            o = b0 + fz * (b1 - b0)
            plsc.store_scatter(out_buf, [r0, colv], o)
            return carry

        lax.fori_loop(i32(0), i32(_NJ), j_body, i32(0))

    def block_body(blk, carry):
        base = wid * i32(per_w) + blk * i32(_N)
        pltpu.sync_copy(x_ref.at[pl.ds(base, _N)], xn_ref.at[pl.ds(0, _N)])
        pltpu.sync_copy(y_ref.at[pl.ds(base, _N)], xn_ref.at[pl.ds(_N, _N)])
        pltpu.sync_copy(z_ref.at[pl.ds(base, _N)], xn_ref.at[pl.ds(2 * _N, _N)])

        def norm_body(i, c):
            s = pl.ds(i * i32(16), 16)
            v = xn_ref[s]
            xn_ref[s] = jnp.clip((v + 1.0) * 0.5, 0.0, 1.0)
            return c

        lax.fori_loop(i32(0), i32(3 * _NCH), norm_body, i32(0))

        bufs = ((fr_a, idx_a, sub_a, rows_a, sem_a),
                (fr_b, idx_b, sub_b, rows_b, sem_b))
        pending = None
        for l in range(6):
            fr, ix, sb, rw, sm = bufs[l % 2]
            pass1(l, fr, ix, sb)
            fire(ix, rw, sm)
            if pending is not None:
                pcol, pix, psb, pfr, prw, psm = pending
                drain(pix, prw, psm)
                pass2(pcol, pfr, psb, prw)
            pending = (cid + i32(2 * l), ix, sb, fr, rw, sm)

        fr0, ix0, sb0, rw0, sm0 = bufs[0]
        fr1, ix1, sb1, rw1, sm1 = bufs[1]

        def lvl_body(k, carry):
            ka = k * i32(2)
            kb = ka + i32(1)
            col_prev = cid + i32(10) + k * i32(4)
            col_a = col_prev + i32(2)
            pass1_u(ka, fr0, ix0, sb0)
            fire(ix0, rw0, sm0)
            drain(ix1, rw1, sm1)
            pass2(col_prev, fr1, sb1, rw1)
            pass1_u(kb, fr1, ix1, sb1)
            fire(ix1, rw1, sm1)
            drain(ix0, rw0, sm0)
            pass2(col_a, fr0, sb0, rw0)
            return carry

        lax.fori_loop(i32(0), i32(5), lvl_body, i32(0))
        drain(ix1, rw1, sm1)
        pass2(cid + i32(30), fr1, sb1, rw1)

        pltpu.sync_copy(out_buf, out_ref.at[pl.ds(base, _N)])
        return carry

    lax.fori_loop(i32(0), i32(nblocks), block_body, i32(0))


def kernel(xyz, tables):
    with jax.enable_x64(False):
        return _kernel_x32(xyz, tables)


def _kernel_x32(xyz, tables):
    x = jnp.asarray(xyz[:, 0], jnp.float32)
    y = jnp.asarray(xyz[:, 1], jnp.float32)
    z = jnp.asarray(xyz[:, 2], jnp.float32)
    mesh = plsc.VectorSubcoreMesh(core_axis_name="c", subcore_axis_name="s")
    scratch = [
        pltpu.HBM((_TOT4, 8), jnp.float32),
        pltpu.VMEM((512, 2), jnp.float32),
        pltpu.VMEM((128, 8), jnp.float32),
        pltpu.VMEM((480,), jnp.int32),
        pltpu.VMEM((160,), jnp.float32),
        pltpu.VMEM((3 * _N,), jnp.float32),
        pltpu.VMEM((3 * _N,), jnp.float32),
        pltpu.VMEM((3 * _N,), jnp.float32),
        pltpu.VMEM((_NDMA, _IDX_CHUNK), jnp.int32),
        pltpu.VMEM((_NDMA, _IDX_CHUNK), jnp.int32),
        pltpu.VMEM((8 * _N,), jnp.int32),
        pltpu.VMEM((8 * _N,), jnp.int32),
        pltpu.VMEM((8 * _N, 8), jnp.float32),
        pltpu.VMEM((8 * _N, 8), jnp.float32),
        pltpu.VMEM((_N, 2 * _NUM_LEVELS), jnp.float32),
        pltpu.SemaphoreType.DMA,
        pltpu.SemaphoreType.DMA,
    ]
    f = pl.kernel(
        _body,
        out_type=jax.ShapeDtypeStruct((_B, 2 * _NUM_LEVELS), jnp.float32),
        mesh=mesh,
        scratch_types=scratch,
        compiler_params=pltpu.CompilerParams(
            needs_layout_passes=False, use_tc_tiling_on_sc=False
        ),
    )
    return f(
        x, y, z,
        *[jnp.asarray(t, jnp.float32) for t in tables],
        jnp.asarray(_CST_I), jnp.asarray(_CST_F),
    )


# speedup vs baseline: 1.4052x; 1.4052x over previous
import numpy as np
import jax
import jax.numpy as jnp
from jax import lax
from jax.experimental import pallas as pl
from jax.experimental.pallas import tpu as pltpu
from jax.experimental.pallas import tpu_sc as plsc

_NUM_LEVELS = 16
_FEAT = 2
_B = 262144
_HS = 2 ** 19
_P2 = 2654435761
_P3 = 805459861

_P2_I32 = np.int32(np.uint32(_P2 & 0xFFFFFFFF).astype(np.int64) - (1 << 32))
_P3_I32 = np.int32(_P3)
_A2 = np.int32(_P2 >> 16)
_B2 = np.int32(_P2 & 0xFFFF)
_A3 = np.int32(_P3 >> 16)
_B3 = np.int32(_P3 & 0xFFFF)
_MININT = np.int32(-2147483648)
_LOW31 = np.int32(0x7FFFFFFF)


def _level_constants():
    b = np.exp2(np.log2(np.float32(2048) / np.float32(16)) / np.float32(15))
    res = (np.float32(16) * (np.float32(b) ** np.arange(16, dtype=np.float32))).astype(np.int32)
    out = []
    for r in res:
        r = int(r)
        m = min(r ** 3, _HS)
        out.append((r, m))
    return out


_LEVELS = _level_constants()
_OFF4 = []
_acc = 0
for _r, _m in _LEVELS:
    _OFF4.append(_acc)
    _acc += _m // 4
_TOT4 = _acc

_MASK19 = np.int32(2 ** 19 - 1)
_CST_I = np.zeros((10, 3, 16), np.int32)
_CST_F = np.zeros((10, 16), np.float32)
for _k, _l in enumerate(range(6, 16)):
    _res = _LEVELS[_l][0]
    _CST_I[_k, 0, :] = _res - 1
    _CST_I[_k, 1, :] = _res
    _CST_I[_k, 2, :] = _OFF4[_l]
    _CST_F[_k, :] = np.float32(_res)
_CST_I = _CST_I.reshape(-1)
_CST_F = _CST_F.reshape(-1)

_N = 512
_NCH = _N // 16
_NJ = _N // 8
_IDX_CHUNK = 128
_NDMA = 8 * _N // _IDX_CHUNK


def _prod64(v, a16, b16):
    u = v * a16
    w = v * b16
    lo = (u << 16) + w
    carry = jnp.where((lo ^ _MININT) < (w ^ _MININT), np.int32(1), np.int32(0))
    hi = (u >> 16) + carry
    return hi, lo


def _body(x_ref, y_ref, z_ref, packed, cst_i_hbm, cst_f_hbm, *rest):
    out_ref = rest[0]
    (ci_ref, cf_ref, xn_ref, fr_a, fr_b, idx_a, idx_b,
     sub_a, sub_b, rows_a, rows_b, out_buf, sem_a, sem_b) = rest[1:]

    nw = 32
    wid = (lax.axis_index("s") * np.int32(2) + lax.axis_index("c")).astype(jnp.int32)
    per_w = _B // nw
    nblocks = per_w // _N
    i32 = np.int32

    iota = lax.iota(jnp.int32, 16)
    ioh = iota >> 1
    cid = iota & 1

    pltpu.sync_copy(cst_i_hbm, ci_ref)
    pltpu.sync_copy(cst_f_hbm, cf_ref)

    def pass1(l, fr_ref, idx_ref, sub_ref):
        res, m = _LEVELS[l]
        pow2 = (m & (m - 1)) == 0
        res_f = np.float32(res)
        res_m1 = np.int32(res - 1)
        res_i = np.int32(res)
        if not pow2:
            r32 = np.int32((1 << 32) % m)
            r31 = np.int32((1 << 31) % m)
            m_i = np.int32(m)
        mask = np.int32(m - 1)

        def c_body(ci, carry):
            off = ci * i32(16)
            los = []
            his = []
            for d in range(3):
                f = xn_ref[pl.ds(i32(d * _N) + off, 16)]
                xg = f * res_f
                xi = xg.astype(jnp.int32)
                fr_ref[pl.ds(i32(d * _N) + off, 16)] = xg - xi.astype(jnp.float32)
                vf = jnp.minimum(xi, res_m1)
                vc = jnp.minimum(xi + 1, res_i)
                if d == 0:
                    los.append((vf, vc))
                    his.append(None)
                else:
                    a16 = _A2 if d == 1 else _A3
                    b16 = _B2 if d == 1 else _B3
                    p32 = _P2_I32 if d == 1 else _P3_I32
                    if pow2:
                        los.append((vf * p32, vc * p32))
                        his.append(None)
                    else:
                        hf, lf = _prod64(vf, a16, b16)
                        hc, lc = _prod64(vc, a16, b16)
                        los.append((lf, lc))
                        his.append((hf, hc))
            for cn in range(8):
                sx, sy, sz = cn & 1, (cn >> 1) & 1, (cn >> 2) & 1
                lo = los[0][sx] ^ los[1][sy] ^ los[2][sz]
                if pow2:
                    h = lo & mask
                else:
                    hi = his[1][sy] ^ his[2][sz]
                    t = hi * r32 + lax.rem(lo & _LOW31, m_i)
                    t = t + jnp.where(lo < 0, r31, np.int32(0))
                    h = lax.rem(t, m_i)
                row = i32(cn * 4) + (ci >> 3)
                col = (ci & i32(7)) * i32(16)
                idx_ref[row, pl.ds(col, 16)] = (h >> 2) + i32(_OFF4[l])
                sub_ref[pl.ds(i32(cn * _N) + off, 16)] = (h & i32(3)) * i32(2)
            return carry

        lax.fori_loop(i32(0), i32(_NCH), c_body, i32(0))

    def pass1_u(kq, fr_ref, idx_ref, sub_ref):
        resf = cf_ref[pl.ds(kq * i32(16), 16)]
        cb = kq * i32(48)
        resm1 = ci_ref[pl.ds(cb, 16)]
        resi = ci_ref[pl.ds(cb + i32(16), 16)]
        off4 = ci_ref[pl.ds(cb + i32(32), 16)]

        def c_body(ci, carry):
            off = ci * i32(16)
            los = []
            for d in range(3):
                f = xn_ref[pl.ds(i32(d * _N) + off, 16)]
                xg = f * resf
                xi = xg.astype(jnp.int32)
                fr_ref[pl.ds(i32(d * _N) + off, 16)] = xg - xi.astype(jnp.float32)
                vf = jnp.minimum(xi, resm1)
                vc = jnp.minimum(xi + 1, resi)
                if d == 0:
                    los.append((vf, vc))
                else:
                    p32 = _P2_I32 if d == 1 else _P3_I32
                    los.append((vf * p32, vc * p32))
            for cn in range(8):
                sx, sy, sz = cn & 1, (cn >> 1) & 1, (cn >> 2) & 1
                lo = los[0][sx] ^ los[1][sy] ^ los[2][sz]
                h = lo & _MASK19
                row = i32(cn * 4) + (ci >> 3)
                col = (ci & i32(7)) * i32(16)
                idx_ref[row, pl.ds(col, 16)] = (h >> 2) + off4
                sub_ref[pl.ds(i32(cn * _N) + off, 16)] = (h & i32(3)) * i32(2)
            return carry

        lax.fori_loop(i32(0), i32(_NCH), c_body, i32(0))

    def fire(idx_ref, rows_ref, sem):
        def f_body(i, carry):
            pltpu.async_copy(
                packed.at[idx_ref.at[i]],
                rows_ref.at[pl.ds(i * i32(_IDX_CHUNK), _IDX_CHUNK)],
                sem,
            )
            return carry

        lax.fori_loop(i32(0), i32(_NDMA), f_body, i32(0))

    def drain(idx_ref, rows_ref, sem):
        def d_body(i, carry):
            pltpu.make_async_copy(
                packed.at[idx_ref.at[i]],
                rows_ref.at[pl.ds(i * i32(_IDX_CHUNK), _IDX_CHUNK)],
                sem,
            ).wait()
            return carry

        lax.fori_loop(i32(0), i32(_NDMA), d_body, i32(0))

    def pass2(colv, fr_ref, sub_ref, rows_ref):
        def j_body(j, carry):
            r0 = j * i32(8) + ioh
            fx = plsc.load_gather(fr_ref, [r0])
            fy = plsc.load_gather(fr_ref, [r0 + np.int32(_N)])
            fz = plsc.load_gather(fr_ref, [r0 + np.int32(2 * _N)])
            rv = []
            for cn in range(8):
                rc = r0 + np.int32(cn * _N)
                sub = plsc.load_gather(sub_ref, [rc])
                rv.append(plsc.load_gather(rows_ref, [rc, sub + cid]))
            a0 = rv[0] + fx * (rv[1] - rv[0])
            a1 = rv[2] + fx * (rv[3] - rv[2])
            a2 = rv[4] + fx * (rv[5] - rv[4])
            a3 = rv[6] + fx * (rv[7] - rv[6])
            b0 = a0 + fy * (a1 - a0)
            b1 = a2 + fy * (a3 - a2)
            o = b0 + fz * (b1 - b0)
            plsc.store_scatter(out_buf, [r0, colv], o)
            return carry

        lax.fori_loop(i32(0), i32(_NJ), j_body, i32(0))

    def block_body(blk, carry):
        base = wid * i32(per_w) + blk * i32(_N)
        pltpu.sync_copy(x_ref.at[pl.ds(base, _N)], xn_ref.at[pl.ds(0, _N)])
        pltpu.sync_copy(y_ref.at[pl.ds(base, _N)], xn_ref.at[pl.ds(_N, _N)])
        pltpu.sync_copy(z_ref.at[pl.ds(base, _N)], xn_ref.at[pl.ds(2 * _N, _N)])

        def norm_body(i, c):
            s = pl.ds(i * i32(16), 16)
            v = xn_ref[s]
            xn_ref[s] = jnp.clip((v + 1.0) * 0.5, 0.0, 1.0)
            return c

        lax.fori_loop(i32(0), i32(3 * _NCH), norm_body, i32(0))

        bufs = ((fr_a, idx_a, sub_a, rows_a, sem_a),
                (fr_b, idx_b, sub_b, rows_b, sem_b))
        pending = None
        for l in range(6):
            fr, ix, sb, rw, sm = bufs[l % 2]
            pass1(l, fr, ix, sb)
            fire(ix, rw, sm)
            if pending is not None:
                pcol, pix, psb, pfr, prw, psm = pending
                drain(pix, prw, psm)
                pass2(pcol, pfr, psb, prw)
            pending = (cid + i32(2 * l), ix, sb, fr, rw, sm)

        fr0, ix0, sb0, rw0, sm0 = bufs[0]
        fr1, ix1, sb1, rw1, sm1 = bufs[1]

        def lvl_body(k, carry):
            ka = k * i32(2)
            kb = ka + i32(1)
            col_prev = cid + i32(10) + k * i32(4)
            col_a = col_prev + i32(2)
            pass1_u(ka, fr0, ix0, sb0)
            fire(ix0, rw0, sm0)
            drain(ix1, rw1, sm1)
            pass2(col_prev, fr1, sb1, rw1)
            pass1_u(kb, fr1, ix1, sb1)
            fire(ix1, rw1, sm1)
            drain(ix0, rw0, sm0)
            pass2(col_a, fr0, sb0, rw0)
            return carry

        lax.fori_loop(i32(0), i32(5), lvl_body, i32(0))
        drain(ix1, rw1, sm1)
        pass2(cid + i32(30), fr1, sb1, rw1)

        pltpu.sync_copy(out_buf, out_ref.at[pl.ds(base, _N)])
        return carry

    lax.fori_loop(i32(0), i32(nblocks), block_body, i32(0))


def kernel(xyz, tables):
    with jax.enable_x64(False):
        return _kernel_x32(xyz, tables)


def _kernel_x32(xyz, tables):
    x = jnp.asarray(xyz[:, 0], jnp.float32)
    y = jnp.asarray(xyz[:, 1], jnp.float32)
    z = jnp.asarray(xyz[:, 2], jnp.float32)
    mesh = plsc.VectorSubcoreMesh(core_axis_name="c", subcore_axis_name="s")
    scratch = [
        pltpu.VMEM((480,), jnp.int32),
        pltpu.VMEM((160,), jnp.float32),
        pltpu.VMEM((3 * _N,), jnp.float32),
        pltpu.VMEM((3 * _N,), jnp.float32),
        pltpu.VMEM((3 * _N,), jnp.float32),
        pltpu.VMEM((_NDMA, _IDX_CHUNK), jnp.int32),
        pltpu.VMEM((_NDMA, _IDX_CHUNK), jnp.int32),
        pltpu.VMEM((8 * _N,), jnp.int32),
        pltpu.VMEM((8 * _N,), jnp.int32),
        pltpu.VMEM((8 * _N, 8), jnp.float32),
        pltpu.VMEM((8 * _N, 8), jnp.float32),
        pltpu.VMEM((_N, 2 * _NUM_LEVELS), jnp.float32),
        pltpu.SemaphoreType.DMA,
        pltpu.SemaphoreType.DMA,
    ]
    f = pl.kernel(
        _body,
        out_type=jax.ShapeDtypeStruct((_B, 2 * _NUM_LEVELS), jnp.float32),
        mesh=mesh,
        scratch_types=scratch,
        compiler_params=pltpu.CompilerParams(
            needs_layout_passes=False, use_tc_tiling_on_sc=False
        ),
    )
    packed = jnp.concatenate(
        [jnp.asarray(t, jnp.float32).reshape(-1) for t in tables]
    ).reshape(_TOT4, 8)
    return f(x, y, z, packed, jnp.asarray(_CST_I), jnp.asarray(_CST_F))


# speedup vs baseline: 1.4093x; 1.0029x over previous
import numpy as np
import jax
import jax.numpy as jnp
from jax import lax
from jax.experimental import pallas as pl
from jax.experimental.pallas import tpu as pltpu
from jax.experimental.pallas import tpu_sc as plsc

_NUM_LEVELS = 16
_FEAT = 2
_B = 262144
_HS = 2 ** 19
_P2 = 2654435761
_P3 = 805459861

_P2_I32 = np.int32(np.uint32(_P2 & 0xFFFFFFFF).astype(np.int64) - (1 << 32))
_P3_I32 = np.int32(_P3)
_A2 = np.int32(_P2 >> 16)
_B2 = np.int32(_P2 & 0xFFFF)
_A3 = np.int32(_P3 >> 16)
_B3 = np.int32(_P3 & 0xFFFF)
_MININT = np.int32(-2147483648)
_LOW31 = np.int32(0x7FFFFFFF)


def _level_constants():
    b = np.exp2(np.log2(np.float32(2048) / np.float32(16)) / np.float32(15))
    res = (np.float32(16) * (np.float32(b) ** np.arange(16, dtype=np.float32))).astype(np.int32)
    out = []
    for r in res:
        r = int(r)
        m = min(r ** 3, _HS)
        out.append((r, m))
    return out


_LEVELS = _level_constants()
_OFF4 = []
_acc = 0
for _r, _m in _LEVELS:
    _OFF4.append(_acc)
    _acc += _m // 4
_TOT4 = _acc

_MASK19 = np.int32(2 ** 19 - 1)
_CST_I = np.zeros((10, 3, 16), np.int32)
_CST_F = np.zeros((10, 16), np.float32)
for _k, _l in enumerate(range(6, 16)):
    _res = _LEVELS[_l][0]
    _CST_I[_k, 0, :] = _res - 1
    _CST_I[_k, 1, :] = _res
    _CST_I[_k, 2, :] = _OFF4[_l]
    _CST_F[_k, :] = np.float32(_res)
_CST_I = _CST_I.reshape(-1)
_CST_F = _CST_F.reshape(-1)

_N = 512
_NCH = _N // 16
_NJ = _N // 8
_IDX_CHUNK = 128
_NDMA = 8 * _N // _IDX_CHUNK


def _prod64(v, a16, b16):
    u = v * a16
    w = v * b16
    lo = (u << 16) + w
    carry = jnp.where((lo ^ _MININT) < (w ^ _MININT), np.int32(1), np.int32(0))
    hi = (u >> 16) + carry
    return hi, lo


def _body(x_ref, y_ref, z_ref, packed, cst_i_hbm, cst_f_hbm, *rest):
    out_ref = rest[0]
    (ci_ref, cf_ref, xn_ref, fr_a, fr_b, idx_a, idx_b,
     sub_a, sub_b, rows_a, rows_b, out_buf, sem_a, sem_b) = rest[1:]

    nw = 32
    wid = (lax.axis_index("s") * np.int32(2) + lax.axis_index("c")).astype(jnp.int32)
    per_w = _B // nw
    nblocks = per_w // _N
    i32 = np.int32

    iota = lax.iota(jnp.int32, 16)
    ioh = iota >> 1
    cid = iota & 1

    pltpu.sync_copy(cst_i_hbm, ci_ref)
    pltpu.sync_copy(cst_f_hbm, cf_ref)

    def pass1(l, fr_ref, idx_ref, sub_ref):
        res, m = _LEVELS[l]
        pow2 = (m & (m - 1)) == 0
        res_f = np.float32(res)
        res_m1 = np.int32(res - 1)
        res_i = np.int32(res)
        if not pow2:
            r32 = np.int32((1 << 32) % m)
            r31 = np.int32((1 << 31) % m)
            m_i = np.int32(m)
        mask = np.int32(m - 1)

        def c_body(ci, carry):
            off = ci * i32(16)
            los = []
            his = []
            for d in range(3):
                f = xn_ref[pl.ds(i32(d * _N) + off, 16)]
                xg = f * res_f
                xi = xg.astype(jnp.int32)
                fr_ref[pl.ds(i32(d * _N) + off, 16)] = xg - xi.astype(jnp.float32)
                vf = jnp.minimum(xi, res_m1)
                vc = jnp.minimum(xi + 1, res_i)
                if d == 0:
                    los.append((vf, vc))
                    his.append(None)
                else:
                    a16 = _A2 if d == 1 else _A3
                    b16 = _B2 if d == 1 else _B3
                    p32 = _P2_I32 if d == 1 else _P3_I32
                    if pow2:
                        los.append((vf * p32, vc * p32))
                        his.append(None)
                    else:
                        hf, lf = _prod64(vf, a16, b16)
                        hc, lc = _prod64(vc, a16, b16)
                        los.append((lf, lc))
                        his.append((hf, hc))
            for cn in range(8):
                sx, sy, sz = cn & 1, (cn >> 1) & 1, (cn >> 2) & 1
                lo = los[0][sx] ^ los[1][sy] ^ los[2][sz]
                if pow2:
                    h = lo & mask
                else:
                    hi = his[1][sy] ^ his[2][sz]
                    t = hi * r32 + lax.rem(lo & _LOW31, m_i)
                    t = t + jnp.where(lo < 0, r31, np.int32(0))
                    h = lax.rem(t, m_i)
                row = i32(cn * 4) + (ci >> 3)
                col = (ci & i32(7)) * i32(16)
                idx_ref[row, pl.ds(col, 16)] = (h >> 2) + i32(_OFF4[l])
                sub_ref[pl.ds(i32(cn * _N) + off, 16)] = (h & i32(3)) * i32(2)
            return carry

        lax.fori_loop(i32(0), i32(_NCH), c_body, i32(0))

    def pass1_u(kq, fr_ref, idx_ref, sub_ref):
        resf = cf_ref[pl.ds(kq * i32(16), 16)]
        cb = kq * i32(48)
        resm1 = ci_ref[pl.ds(cb, 16)]
        resi = ci_ref[pl.ds(cb + i32(16), 16)]
        off4 = ci_ref[pl.ds(cb + i32(32), 16)]

        def c_body(ci, carry):
            off = ci * i32(16)
            los = []
            for d in range(3):
                f = xn_ref[pl.ds(i32(d * _N) + off, 16)]
                xg = f * resf
                xi = xg.astype(jnp.int32)
                fr_ref[pl.ds(i32(d * _N) + off, 16)] = xg - xi.astype(jnp.float32)
                vf = jnp.minimum(xi, resm1)
                vc = jnp.minimum(xi + 1, resi)
                if d == 0:
                    los.append((vf, vc))
                else:
                    p32 = _P2_I32 if d == 1 else _P3_I32
                    los.append((vf * p32, vc * p32))
            for cn in range(8):
                sx, sy, sz = cn & 1, (cn >> 1) & 1, (cn >> 2) & 1
                lo = los[0][sx] ^ los[1][sy] ^ los[2][sz]
                h = lo & _MASK19
                row = i32(cn * 4) + (ci >> 3)
                col = (ci & i32(7)) * i32(16)
                idx_ref[row, pl.ds(col, 16)] = (h >> 2) + off4
                sub_ref[pl.ds(i32(cn * _N) + off, 16)] = (h & i32(3)) * i32(2)
            return carry

        lax.fori_loop(i32(0), i32(_NCH), c_body, i32(0))

    def fire(idx_ref, rows_ref, sem):
        def f_body(i, carry):
            pltpu.async_copy(
                packed.at[idx_ref.at[i]],
                rows_ref.at[pl.ds(i * i32(_IDX_CHUNK), _IDX_CHUNK)],
                sem,
            )
            return carry

        lax.fori_loop(i32(0), i32(_NDMA), f_body, i32(0))

    def drain(idx_ref, rows_ref, sem):
        def d_body(i, carry):
            pltpu.make_async_copy(
                packed.at[idx_ref.at[i]],
                rows_ref.at[pl.ds(i * i32(_IDX_CHUNK), _IDX_CHUNK)],
                sem,
            ).wait()
            return carry

        lax.fori_loop(i32(0), i32(_NDMA), d_body, i32(0))

    def pass2(colv, fr_ref, sub_ref, rows_ref):
        def j_body(j, carry):
            r0 = j * i32(8) + ioh
            fx = plsc.load_gather(fr_ref, [r0])
            fy = plsc.load_gather(fr_ref, [r0 + np.int32(_N)])
            fz = plsc.load_gather(fr_ref, [r0 + np.int32(2 * _N)])
            rv = []
            for cn in range(8):
                rc = r0 + np.int32(cn * _N)
                sub = plsc.load_gather(sub_ref, [rc])
                rv.append(plsc.load_gather(rows_ref, [rc, sub + cid]))
            a0 = rv[0] + fx * (rv[1] - rv[0])
            a1 = rv[2] + fx * (rv[3] - rv[2])
            a2 = rv[4] + fx * (rv[5] - rv[4])
            a3 = rv[6] + fx * (rv[7] - rv[6])
            b0 = a0 + fy * (a1 - a0)
            b1 = a2 + fy * (a3 - a2)
            o = b0 + fz * (b1 - b0)
            plsc.store_scatter(out_buf, [r0, colv], o)
            return carry

        lax.fori_loop(i32(0), i32(_NJ), j_body, i32(0))

    def block_body(blk, carry):
        base = wid * i32(per_w) + blk * i32(_N)
        pltpu.sync_copy(x_ref.at[pl.ds(base, _N)], xn_ref.at[pl.ds(0, _N)])
        pltpu.sync_copy(y_ref.at[pl.ds(base, _N)], xn_ref.at[pl.ds(_N, _N)])
        pltpu.sync_copy(z_ref.at[pl.ds(base, _N)], xn_ref.at[pl.ds(2 * _N, _N)])

        def norm_body(i, c):
            s = pl.ds(i * i32(16), 16)
            v = xn_ref[s]
            xn_ref[s] = jnp.clip((v + 1.0) * 0.5, 0.0, 1.0)
            return c

        lax.fori_loop(i32(0), i32(3 * _NCH), norm_body, i32(0))

        bufs = ((fr_a, idx_a, sub_a, rows_a, sem_a),
                (fr_b, idx_b, sub_b, rows_b, sem_b))
        pending = None
        for l in range(_NUM_LEVELS):
            fr, ix, sb, rw, sm = bufs[l % 2]
            pass1(l, fr, ix, sb)
            fire(ix, rw, sm)
            if pending is not None:
                pcol, pix, psb, pfr, prw, psm = pending
                drain(pix, prw, psm)
                pass2(pcol, pfr, psb, prw)
            pending = (cid + i32(2 * l), ix, sb, fr, rw, sm)
        pcol, pix, psb, pfr, prw, psm = pending
        drain(pix, prw, psm)
        pass2(pcol, pfr, psb, prw)

        pltpu.sync_copy(out_buf, out_ref.at[pl.ds(base, _N)])
        return carry

    lax.fori_loop(i32(0), i32(nblocks), block_body, i32(0))


def kernel(xyz, tables):
    with jax.enable_x64(False):
        return _kernel_x32(xyz, tables)


def _kernel_x32(xyz, tables):
    x = jnp.asarray(xyz[:, 0], jnp.float32)
    y = jnp.asarray(xyz[:, 1], jnp.float32)
    z = jnp.asarray(xyz[:, 2], jnp.float32)
    mesh = plsc.VectorSubcoreMesh(core_axis_name="c", subcore_axis_name="s")
    scratch = [
        pltpu.VMEM((480,), jnp.int32),
        pltpu.VMEM((160,), jnp.float32),
        pltpu.VMEM((3 * _N,), jnp.float32),
        pltpu.VMEM((3 * _N,), jnp.float32),
        pltpu.VMEM((3 * _N,), jnp.float32),
        pltpu.VMEM((_NDMA, _IDX_CHUNK), jnp.int32),
        pltpu.VMEM((_NDMA, _IDX_CHUNK), jnp.int32),
        pltpu.VMEM((8 * _N,), jnp.int32),
        pltpu.VMEM((8 * _N,), jnp.int32),
        pltpu.VMEM((8 * _N, 8), jnp.float32),
        pltpu.VMEM((8 * _N, 8), jnp.float32),
        pltpu.VMEM((_N, 2 * _NUM_LEVELS), jnp.float32),
        pltpu.SemaphoreType.DMA,
        pltpu.SemaphoreType.DMA,
    ]
    f = pl.kernel(
        _body,
        out_type=jax.ShapeDtypeStruct((_B, 2 * _NUM_LEVELS), jnp.float32),
        mesh=mesh,
        scratch_types=scratch,
        compiler_params=pltpu.CompilerParams(
            needs_layout_passes=False, use_tc_tiling_on_sc=False
        ),
    )
    packed = jnp.concatenate(
        [jnp.asarray(t, jnp.float32).reshape(-1) for t in tables]
    ).reshape(_TOT4, 8)
    return f(x, y, z, packed, jnp.asarray(_CST_I), jnp.asarray(_CST_F))


# speedup vs baseline: 1.5184x; 1.0774x over previous
import numpy as np
import jax
import jax.numpy as jnp
from jax import lax
from jax.experimental import pallas as pl
from jax.experimental.pallas import tpu as pltpu
from jax.experimental.pallas import tpu_sc as plsc

_NUM_LEVELS = 16
_FEAT = 2
_B = 262144
_HS = 2 ** 19
_P2 = 2654435761
_P3 = 805459861

_P2_I32 = np.int32(np.uint32(_P2 & 0xFFFFFFFF).astype(np.int64) - (1 << 32))
_P3_I32 = np.int32(_P3)
_A2 = np.int32(_P2 >> 16)
_B2 = np.int32(_P2 & 0xFFFF)
_A3 = np.int32(_P3 >> 16)
_B3 = np.int32(_P3 & 0xFFFF)
_MININT = np.int32(-2147483648)
_LOW31 = np.int32(0x7FFFFFFF)


def _level_constants():
    b = np.exp2(np.log2(np.float32(2048) / np.float32(16)) / np.float32(15))
    res = (np.float32(16) * (np.float32(b) ** np.arange(16, dtype=np.float32))).astype(np.int32)
    out = []
    for r in res:
        r = int(r)
        m = min(r ** 3, _HS)
        out.append((r, m))
    return out


_LEVELS = _level_constants()
_OFF4 = []
_acc = 0
for _r, _m in _LEVELS:
    _OFF4.append(_acc)
    _acc += _m // 4
_TOT4 = _acc

_MASK19 = np.int32(2 ** 19 - 1)
_CST_I = np.zeros((10, 3, 16), np.int32)
_CST_F = np.zeros((10, 16), np.float32)
for _k, _l in enumerate(range(6, 16)):
    _res = _LEVELS[_l][0]
    _CST_I[_k, 0, :] = _res - 1
    _CST_I[_k, 1, :] = _res
    _CST_I[_k, 2, :] = _OFF4[_l]
    _CST_F[_k, :] = np.float32(_res)
_CST_I = _CST_I.reshape(-1)
_CST_F = _CST_F.reshape(-1)

_N = 512
_NCH = _N // 16
_NJ = _N // 8
_IDX_CHUNK = 128
_NDMA = 8 * _N // _IDX_CHUNK


def _prod64(v, a16, b16):
    u = v * a16
    w = v * b16
    lo = (u << 16) + w
    carry = jnp.where((lo ^ _MININT) < (w ^ _MININT), np.int32(1), np.int32(0))
    hi = (u >> 16) + carry
    return hi, lo


def _body(x_ref, y_ref, z_ref, *rest):
    tables = rest[:_NUM_LEVELS]
    out_ref = rest[_NUM_LEVELS]
    (xn_ref, fr_a, fr_b, idx_a, idx_b,
     sub_a, sub_b, rows_a, rows_b, out_buf, sem_a, sem_b) = rest[_NUM_LEVELS + 1:]

    nw = 32
    wid = (lax.axis_index("s") * np.int32(2) + lax.axis_index("c")).astype(jnp.int32)
    per_w = _B // nw
    nblocks = per_w // _N
    i32 = np.int32

    iota = lax.iota(jnp.int32, 16)
    ioh = iota >> 1
    cid = iota & 1

    def pass1(l, fr_ref, idx_ref, sub_ref):
        res, m = _LEVELS[l]
        pow2 = (m & (m - 1)) == 0
        res_f = np.float32(res)
        res_m1 = np.int32(res - 1)
        res_i = np.int32(res)
        if not pow2:
            r32 = np.int32((1 << 32) % m)
            r31 = np.int32((1 << 31) % m)
            m_i = np.int32(m)
        mask = np.int32(m - 1)

        def c_body(ci, carry):
            off = ci * i32(16)
            los = []
            his = []
            for d in range(3):
                f = xn_ref[pl.ds(i32(d * _N) + off, 16)]
                xg = f * res_f
                xi = xg.astype(jnp.int32)
                fr_ref[pl.ds(i32(d * _N) + off, 16)] = xg - xi.astype(jnp.float32)
                vf = jnp.minimum(xi, res_m1)
                vc = jnp.minimum(xi + 1, res_i)
                if d == 0:
                    los.append((vf, vc))
                    his.append(None)
                else:
                    a16 = _A2 if d == 1 else _A3
                    b16 = _B2 if d == 1 else _B3
                    p32 = _P2_I32 if d == 1 else _P3_I32
                    if pow2:
                        los.append((vf * p32, vc * p32))
                        his.append(None)
                    else:
                        hf, lf = _prod64(vf, a16, b16)
                        hc, lc = _prod64(vc, a16, b16)
                        los.append((lf, lc))
                        his.append((hf, hc))
            for cn in range(8):
                sx, sy, sz = cn & 1, (cn >> 1) & 1, (cn >> 2) & 1
                lo = los[0][sx] ^ los[1][sy] ^ los[2][sz]
                if pow2:
                    h = lo & mask
                else:
                    hi = his[1][sy] ^ his[2][sz]
                    t = hi * r32 + lax.rem(lo & _LOW31, m_i)
                    t = t + jnp.where(lo < 0, r31, np.int32(0))
                    h = lax.rem(t, m_i)
                row = i32(cn * 4) + (ci >> 3)
                col = (ci & i32(7)) * i32(16)
                idx_ref[row, pl.ds(col, 16)] = h >> 2
                sub_ref[pl.ds(i32(cn * _N) + off, 16)] = (h & i32(3)) * i32(2)
            return carry

        lax.fori_loop(i32(0), i32(_NCH), c_body, i32(0))

    def fire(tab, idx_ref, rows_ref, sem):
        def f_body(i, carry):
            pltpu.async_copy(
                tab.at[idx_ref.at[i]],
                rows_ref.at[pl.ds(i * i32(_IDX_CHUNK), _IDX_CHUNK)],
                sem,
            )
            return carry

        lax.fori_loop(i32(0), i32(_NDMA), f_body, i32(0))

    def drain(tab, idx_ref, rows_ref, sem):
        def d_body(i, carry):
            pltpu.make_async_copy(
                tab.at[idx_ref.at[i]],
                rows_ref.at[pl.ds(i * i32(_IDX_CHUNK), _IDX_CHUNK)],
                sem,
            ).wait()
            return carry

        lax.fori_loop(i32(0), i32(_NDMA), d_body, i32(0))

    def pass2(colv, fr_ref, sub_ref, rows_ref):
        def j_body(j, carry):
            r0 = j * i32(8) + ioh
            fx = plsc.load_gather(fr_ref, [r0])
            fy = plsc.load_gather(fr_ref, [r0 + np.int32(_N)])
            fz = plsc.load_gather(fr_ref, [r0 + np.int32(2 * _N)])
            rv = []
            for cn in range(8):
                rc = r0 + np.int32(cn * _N)
                sub = plsc.load_gather(sub_ref, [rc])
                rv.append(plsc.load_gather(rows_ref, [rc, sub + cid]))
            a0 = rv[0] + fx * (rv[1] - rv[0])
            a1 = rv[2] + fx * (rv[3] - rv[2])
            a2 = rv[4] + fx * (rv[5] - rv[4])
            a3 = rv[6] + fx * (rv[7] - rv[6])
            b0 = a0 + fy * (a1 - a0)
            b1 = a2 + fy * (a3 - a2)
            o = b0 + fz * (b1 - b0)
            plsc.store_scatter(out_buf, [r0, colv], o)
            return carry

        lax.fori_loop(i32(0), i32(_NJ), j_body, i32(0))

    def block_body(blk, carry):
        base = wid * i32(per_w) + blk * i32(_N)
        pltpu.sync_copy(x_ref.at[pl.ds(base, _N)], xn_ref.at[pl.ds(0, _N)])
        pltpu.sync_copy(y_ref.at[pl.ds(base, _N)], xn_ref.at[pl.ds(_N, _N)])
        pltpu.sync_copy(z_ref.at[pl.ds(base, _N)], xn_ref.at[pl.ds(2 * _N, _N)])

        def norm_body(i, c):
            s = pl.ds(i * i32(16), 16)
            v = xn_ref[s]
            xn_ref[s] = jnp.clip((v + 1.0) * 0.5, 0.0, 1.0)
            return c

        lax.fori_loop(i32(0), i32(3 * _NCH), norm_body, i32(0))

        bufs = ((fr_a, idx_a, sub_a, rows_a, sem_a),
                (fr_b, idx_b, sub_b, rows_b, sem_b))
        pending = None
        for l in range(_NUM_LEVELS):
            fr, ix, sb, rw, sm = bufs[l % 2]
            pass1(l, fr, ix, sb)
            fire(tables[l], ix, rw, sm)
            if pending is not None:
                ptab, pcol, pix, psb, pfr, prw, psm = pending
                drain(ptab, pix, prw, psm)
                pass2(pcol, pfr, psb, prw)
            pending = (tables[l], cid + i32(2 * l), ix, sb, fr, rw, sm)
        ptab, pcol, pix, psb, pfr, prw, psm = pending
        drain(ptab, pix, prw, psm)
        pass2(pcol, pfr, psb, prw)

        pltpu.sync_copy(out_buf, out_ref.at[pl.ds(base, _N)])
        return carry

    lax.fori_loop(i32(0), i32(nblocks), block_body, i32(0))


def kernel(xyz, tables):
    with jax.enable_x64(False):
        return _kernel_x32(xyz, tables)


def _kernel_x32(xyz, tables):
    x = jnp.asarray(xyz[:, 0], jnp.float32)
    y = jnp.asarray(xyz[:, 1], jnp.float32)
    z = jnp.asarray(xyz[:, 2], jnp.float32)
    mesh = plsc.VectorSubcoreMesh(core_axis_name="c", subcore_axis_name="s")
    scratch = [
        pltpu.VMEM((3 * _N,), jnp.float32),
        pltpu.VMEM((3 * _N,), jnp.float32),
        pltpu.VMEM((3 * _N,), jnp.float32),
        pltpu.VMEM((_NDMA, _IDX_CHUNK), jnp.int32),
        pltpu.VMEM((_NDMA, _IDX_CHUNK), jnp.int32),
        pltpu.VMEM((8 * _N,), jnp.int32),
        pltpu.VMEM((8 * _N,), jnp.int32),
        pltpu.VMEM((8 * _N, 8), jnp.float32),
        pltpu.VMEM((8 * _N, 8), jnp.float32),
        pltpu.VMEM((_N, 2 * _NUM_LEVELS), jnp.float32),
        pltpu.SemaphoreType.DMA,
        pltpu.SemaphoreType.DMA,
    ]
    f = pl.kernel(
        _body,
        out_type=jax.ShapeDtypeStruct((_B, 2 * _NUM_LEVELS), jnp.float32),
        mesh=mesh,
        scratch_types=scratch,
        compiler_params=pltpu.CompilerParams(
            needs_layout_passes=False, use_tc_tiling_on_sc=False
        ),
    )
    packed = [
        jnp.asarray(t, jnp.float32).reshape(t.shape[0] // 4, 8) for t in tables
    ]
    return f(x, y, z, *packed)


# speedup vs baseline: 1.5187x; 1.0002x over previous
import numpy as np
import jax
import jax.numpy as jnp
from jax import lax
from jax.experimental import pallas as pl
from jax.experimental.pallas import tpu as pltpu
from jax.experimental.pallas import tpu_sc as plsc

_NUM_LEVELS = 16
_FEAT = 2
_B = 262144
_HS = 2 ** 19
_P2 = 2654435761
_P3 = 805459861

_P2_I32 = np.int32(np.uint32(_P2 & 0xFFFFFFFF).astype(np.int64) - (1 << 32))
_P3_I32 = np.int32(_P3)
_A2 = np.int32(_P2 >> 16)
_B2 = np.int32(_P2 & 0xFFFF)
_A3 = np.int32(_P3 >> 16)
_B3 = np.int32(_P3 & 0xFFFF)
_MININT = np.int32(-2147483648)
_LOW31 = np.int32(0x7FFFFFFF)


def _level_constants():
    b = np.exp2(np.log2(np.float32(2048) / np.float32(16)) / np.float32(15))
    res = (np.float32(16) * (np.float32(b) ** np.arange(16, dtype=np.float32))).astype(np.int32)
    out = []
    for r in res:
        r = int(r)
        m = min(r ** 3, _HS)
        out.append((r, m))
    return out


_LEVELS = _level_constants()

_N = 512
_NCH = _N // 16
_NJ = _N // 8
_IDX_CHUNK = 128
_NDMA = 8 * _N // _IDX_CHUNK


def _prod64(v, a16, b16):
    u = v * a16
    w = v * b16
    lo = (u << 16) + w
    carry = jnp.where((lo ^ _MININT) < (w ^ _MININT), np.int32(1), np.int32(0))
    hi = (u >> 16) + carry
    return hi, lo


def _body(x_ref, y_ref, z_ref, *rest):
    tables = rest[:_NUM_LEVELS]
    out_ref = rest[_NUM_LEVELS]
    (xn_ref, fr_a, fr_b, idx_a, idx_b,
     sub_a, sub_b, rows_a, rows_b, out_buf, sem_a, sem_b) = rest[_NUM_LEVELS + 1:]

    nw = 32
    wid = (lax.axis_index("s") * np.int32(2) + lax.axis_index("c")).astype(jnp.int32)
    per_w = _B // nw
    nblocks = per_w // _N
    i32 = np.int32

    iota = lax.iota(jnp.int32, 16)
    ioh = iota >> 1
    cid = iota & 1

    def pass1(l, fr_ref, idx_ref, sub_ref):
        res, m = _LEVELS[l]
        pow2 = (m & (m - 1)) == 0
        res_f = np.float32(res)
        res_m1 = np.int32(res - 1)
        res_i = np.int32(res)
        if not pow2:
            r32 = np.int32((1 << 32) % m)
            r31 = np.int32((1 << 31) % m)
            m_i = np.int32(m)
        mask = np.int32(m - 1)

        def c_body(ci, carry):
            off = ci * i32(16)
            los = []
            his = []
            for d in range(3):
                f = xn_ref[pl.ds(i32(d * _N) + off, 16)]
                xg = f * res_f
                xi = xg.astype(jnp.int32)
                fr_ref[pl.ds(i32(d * _N) + off, 16)] = xg - xi.astype(jnp.float32)
                vf = jnp.minimum(xi, res_m1)
                vc = jnp.minimum(xi + 1, res_i)
                if d == 0:
                    los.append((vf, vc))
                    his.append(None)
                else:
                    a16 = _A2 if d == 1 else _A3
                    b16 = _B2 if d == 1 else _B3
                    p32 = _P2_I32 if d == 1 else _P3_I32
                    if pow2:
                        los.append((vf * p32, vc * p32))
                        his.append(None)
                    else:
                        hf, lf = _prod64(vf, a16, b16)
                        hc, lc = _prod64(vc, a16, b16)
                        los.append((lf, lc))
                        his.append((hf, hc))
            for cn in range(8):
                sx, sy, sz = cn & 1, (cn >> 1) & 1, (cn >> 2) & 1
                lo = los[0][sx] ^ los[1][sy] ^ los[2][sz]
                if pow2:
                    h = lo & mask
                else:
                    hi = his[1][sy] ^ his[2][sz]
                    t = hi * r32 + lax.rem(lo & _LOW31, m_i)
                    t = t + jnp.where(lo < 0, r31, np.int32(0))
                    h = lax.rem(t, m_i)
                row = i32(cn * 4) + (ci >> 3)
                col = (ci & i32(7)) * i32(16)
                idx_ref[row, pl.ds(col, 16)] = h >> 2
                sub_ref[pl.ds(i32(cn * _N) + off, 16)] = (h & i32(3)) * i32(2)
            return carry

        lax.fori_loop(i32(0), i32(_NCH), c_body, i32(0))

    def fire(tab, idx_ref, rows_ref, sem):
        def f_body(i, carry):
            pltpu.async_copy(
                tab.at[idx_ref.at[i]],
                rows_ref.at[pl.ds(i * i32(_IDX_CHUNK), _IDX_CHUNK)],
                sem,
            )
            return carry

        lax.fori_loop(i32(0), i32(_NDMA), f_body, i32(0))

    def drain(tab, idx_ref, rows_ref, sem):
        def d_body(i, carry):
            pltpu.make_async_copy(
                tab.at[idx_ref.at[i]],
                rows_ref.at[pl.ds(i * i32(_IDX_CHUNK), _IDX_CHUNK)],
                sem,
            ).wait()
            return carry

        lax.fori_loop(i32(0), i32(_NDMA), d_body, i32(0))

    def pass2(colv, fr_ref, sub_ref, rows_ref):
        def j_body(j, carry):
            r0 = j * i32(8) + ioh
            fx = plsc.load_gather(fr_ref, [r0])
            fy = plsc.load_gather(fr_ref, [r0 + np.int32(_N)])
            fz = plsc.load_gather(fr_ref, [r0 + np.int32(2 * _N)])
            rv = []
            for cn in range(8):
                rc = r0 + np.int32(cn * _N)
                sub = plsc.load_gather(sub_ref, [rc])
                rv.append(plsc.load_gather(rows_ref, [rc, sub + cid]))
            a0 = rv[0] + fx * (rv[1] - rv[0])
            a1 = rv[2] + fx * (rv[3] - rv[2])
            a2 = rv[4] + fx * (rv[5] - rv[4])
            a3 = rv[6] + fx * (rv[7] - rv[6])
            b0 = a0 + fy * (a1 - a0)
            b1 = a2 + fy * (a3 - a2)
            o = b0 + fz * (b1 - b0)
            plsc.store_scatter(out_buf, [r0, colv], o)
            return carry

        lax.fori_loop(i32(0), i32(_NJ), j_body, i32(0))

    def block_body(blk, carry):
        base = wid * i32(per_w) + blk * i32(_N)
        pltpu.sync_copy(x_ref.at[pl.ds(base, _N)], xn_ref.at[pl.ds(0, _N)])
        pltpu.sync_copy(y_ref.at[pl.ds(base, _N)], xn_ref.at[pl.ds(_N, _N)])
        pltpu.sync_copy(z_ref.at[pl.ds(base, _N)], xn_ref.at[pl.ds(2 * _N, _N)])

        def norm_body(i, c):
            s = pl.ds(i * i32(16), 16)
            v = xn_ref[s]
            xn_ref[s] = jnp.clip((v + 1.0) * 0.5, 0.0, 1.0)
            return c

        lax.fori_loop(i32(0), i32(3 * _NCH), norm_body, i32(0))

        bufs = ((fr_a, idx_a, sub_a, rows_a, sem_a),
                (fr_b, idx_b, sub_b, rows_b, sem_b))
        pending = None
        for l in range(_NUM_LEVELS):
            fr, ix, sb, rw, sm = bufs[l % 2]
            pass1(l, fr, ix, sb)
            fire(tables[l], ix, rw, sm)
            if pending is not None:
                ptab, pcol, pix, psb, pfr, prw, psm = pending
                drain(ptab, pix, prw, psm)
                pass2(pcol, pfr, psb, prw)
            pending = (tables[l], cid + i32(2 * l), ix, sb, fr, rw, sm)
        ptab, pcol, pix, psb, pfr, prw, psm = pending
        drain(ptab, pix, prw, psm)
        pass2(pcol, pfr, psb, prw)

        pltpu.sync_copy(out_buf, out_ref.at[pl.ds(base, _N)])
        return carry

    lax.fori_loop(i32(0), i32(nblocks), block_body, i32(0))


def kernel(xyz, tables):
    with jax.enable_x64(False):
        return _kernel_x32(xyz, tables)


def _kernel_x32(xyz, tables):
    x = jnp.asarray(xyz[:, 0], jnp.float32)
    y = jnp.asarray(xyz[:, 1], jnp.float32)
    z = jnp.asarray(xyz[:, 2], jnp.float32)
    mesh = plsc.VectorSubcoreMesh(core_axis_name="c", subcore_axis_name="s")
    scratch = [
        pltpu.VMEM((3 * _N,), jnp.float32),
        pltpu.VMEM((3 * _N,), jnp.float32),
        pltpu.VMEM((3 * _N,), jnp.float32),
        pltpu.VMEM((_NDMA, _IDX_CHUNK), jnp.int32),
        pltpu.VMEM((_NDMA, _IDX_CHUNK), jnp.int32),
        pltpu.VMEM((8 * _N,), jnp.int32),
        pltpu.VMEM((8 * _N,), jnp.int32),
        pltpu.VMEM((8 * _N, 8), jnp.float32),
        pltpu.VMEM((8 * _N, 8), jnp.float32),
        pltpu.VMEM((_N, 2 * _NUM_LEVELS), jnp.float32),
        pltpu.SemaphoreType.DMA,
        pltpu.SemaphoreType.DMA,
    ]
    f = pl.kernel(
        _body,
        out_type=jax.ShapeDtypeStruct((_B, 2 * _NUM_LEVELS), jnp.float32),
        mesh=mesh,
        scratch_types=scratch,
        compiler_params=pltpu.CompilerParams(
            needs_layout_passes=False, use_tc_tiling_on_sc=False
        ),
    )
    packed = [
        jnp.asarray(t, jnp.float32).reshape(t.shape[0] // 4, 8) for t in tables
    ]
    return f(x, y, z, *packed)
